# baseline (speedup 1.0000x reference)
import jax
import jax.numpy as jnp
from jax import lax
from jax.experimental import pallas as pl
from jax.experimental.pallas import tpu as pltpu

_C = 256
_LEVELS = [(128, 128), (64, 64), (32, 32)]
_CC = 32
_CH = 2048
_N0 = 128 * 128

_F32 = jnp.float32
_BF = jnp.bfloat16


def _level(H, W, f_ref, w1t, b1, w2t, b2, gwt, gbs, hw1t, hb1, hw2, hb2,
           out_ref, A, B):
    N = H * W
    ch = min(_CH, N)
    logw = W.bit_length() - 1

    n = lax.broadcasted_iota(jnp.int32, (1, N), 1)
    col = n & (W - 1)
    row = lax.shift_right_logical(n, logw)
    top = (row == 0)
    bot = (row == H - 1)
    lft = (col == 0)
    rgt = (col == W - 1)
    deg = (5.0 - top.astype(_F32) - bot.astype(_F32)
           - lft.astype(_F32) - rgt.astype(_F32))
    dinv = lax.rsqrt(deg)
    mU = 1.0 - top.astype(_F32)
    mD = 1.0 - bot.astype(_F32)
    mL = 1.0 - lft.astype(_F32)
    mR = 1.0 - rgt.astype(_F32)

    def roll(x, k):
        return pltpu.roll(x, k % N, 1)

    for n0 in range(0, N, ch):
        sl = slice(n0, n0 + ch)
        x1 = jnp.maximum(
            jnp.dot(w1t, f_ref[:, sl], preferred_element_type=_F32)
            + b1[:], 0.0)
        A[:, sl] = jnp.dot(w2t, x1.astype(_BF),
                           preferred_element_type=_F32) + b2[:]

    layers = ((0, A, B), (1, B, A), (2, A, B))
    for i, src, dst in layers:
        gb = gbs[i]
        for n0 in range(0, N, ch):
            sl = slice(n0, n0 + ch)
            src[:, sl] = jnp.dot(gwt[i],
                                 (src[:, sl] * dinv[:, sl]).astype(_BF),
                                 preferred_element_type=_F32)
        for c0 in range(0, _C, _CC):
            cs = slice(c0, c0 + _CC)
            g = src[cs, :]
            agg = (g
                   + mU * roll(g, W) + mD * roll(g, -W)
                   + mL * roll(g, 1) + mR * roll(g, -1))
            dst[cs, :] = dinv * agg + gb[cs, :]

    for n0 in range(0, N, ch):
        sl = slice(n0, n0 + ch)
        t = jnp.maximum(
            jnp.dot(hw1t, B[:, sl].astype(_BF),
                    preferred_element_type=_F32) + hb1[:],
            0.0)
        out_ref[:, sl] = jnp.dot(hw2[:].astype(_BF), t.astype(_BF),
                                 preferred_element_type=_F32) + hb2[:]


def _body(f0, f1, f2,
          e0w1, e0b1, e0w2, e0b2,
          e1w1, e1b1, e1w2, e1b2,
          e2w1, e2b1, e2w2, e2b2,
          gw0, gb0, gw1, gb1, gw2, gb2,
          hw1, hb1, hw2, hb2,
          o0, o1, o2, A, B, WT):
    ws = (e0w1, e0w2, e1w1, e1w2, e2w1, e2w2, gw0, gw1, gw2, hw1)
    for i, w in enumerate(ws):
        WT[i, :, :] = w[:].T.astype(_BF)
    gwt = (WT[6], WT[7], WT[8])
    gbs = (gb0, gb1, gb2)
    hw1t = WT[9]
    enc = ((WT[0], e0b1, WT[1], e0b2), (WT[2], e1b1, WT[3], e1b2),
           (WT[4], e2b1, WT[5], e2b2))
    for (H, W), f, o, (w1t, b1, w2t, b2) in zip(_LEVELS, (f0, f1, f2),
                                                (o0, o1, o2), enc):
        N = H * W
        a = A.at[:, :N] if N < _N0 else A
        b = B.at[:, :N] if N < _N0 else B
        _level(H, W, f, w1t, b1, w2t, b2, gwt, gbs, hw1t, hb1, hw2, hb2,
               o, a, b)


@jax.jit
def kernel(feat0, feat1, feat2, e0W1, e0b1, e0W2, e0b2, e1W1, e1b1, e1W2,
           e1b2, e2W1, e2b1, e2W2, e2b2, g0W, g0b, g1W, g1b, g2W, g2b,
           hW1, hb1, hW2, hb2):
    c1 = lambda v: v.reshape(_C, 1)
    args = (
        feat0.reshape(_C, 128 * 128).astype(_BF),
        feat1.reshape(_C, 64 * 64).astype(_BF),
        feat2.reshape(_C, 32 * 32).astype(_BF),
        e0W1, c1(e0b1), e0W2, c1(e0b2),
        e1W1, c1(e1b1), e1W2, c1(e1b2),
        e2W1, c1(e2b1), e2W2, c1(e2b2),
        g0W, c1(g0b), g1W, c1(g1b), g2W, c1(g2b),
        hW1, c1(hb1), hW2.reshape(1, _C), hb2.reshape(1, 1),
    )
    out_shapes = tuple(jax.ShapeDtypeStruct((1, H * W), _F32)
                       for H, W in _LEVELS)
    outs = pl.pallas_call(
        _body,
        out_shape=out_shapes,
        scratch_shapes=[pltpu.VMEM((_C, _N0), _F32),
                        pltpu.VMEM((_C, _N0), _F32),
                        pltpu.VMEM((10, _C, _C), _BF)],
    )(*args)
    return tuple(o.reshape(1, H, W, 1)
                 for o, (H, W) in zip(outs, _LEVELS))


# speedup vs baseline: 1.2176x; 1.2176x over previous
import jax
import jax.numpy as jnp
from jax import lax
from jax.experimental import pallas as pl
from jax.experimental.pallas import tpu as pltpu

_C = 256
_LEVELS = [(128, 128), (64, 64), (32, 32)]
_CC = 32
_CH = 2048
_N0 = 128 * 128

_F32 = jnp.float32
_BF = jnp.bfloat16


def _level(H, W, f_ref, w1t, b1, w2t, b2, gwt, gbs, hw1t, hb1, hw2, hb2,
           out_ref, A, B):
    N = H * W
    ch = min(_CH, N)
    logw = W.bit_length() - 1

    n = lax.broadcasted_iota(jnp.int32, (1, N), 1)
    col = n & (W - 1)
    row = lax.shift_right_logical(n, logw)
    top = (row == 0)
    bot = (row == H - 1)
    lft = (col == 0)
    rgt = (col == W - 1)
    deg = (5.0 - top.astype(_F32) - bot.astype(_F32)
           - lft.astype(_F32) - rgt.astype(_F32))
    dinv = lax.rsqrt(deg)
    dinv_b = dinv.astype(_BF)
    mU = (1.0 - top.astype(_F32)).astype(_BF)
    mD = (1.0 - bot.astype(_F32)).astype(_BF)
    mL = (1.0 - lft.astype(_F32)).astype(_BF)
    mR = (1.0 - rgt.astype(_F32)).astype(_BF)

    def roll(x, k):
        return pltpu.roll(x, k % N, 1)

    for n0 in range(0, N, ch):
        sl = slice(n0, n0 + ch)
        x1 = jnp.maximum(
            jnp.dot(w1t, f_ref[:, sl], preferred_element_type=_F32)
            + b1[:], 0.0)
        A[:, sl] = (jnp.dot(w2t, x1.astype(_BF),
                            preferred_element_type=_F32)
                    + b2[:]).astype(_BF)

    layers = ((0, A, B), (1, B, A), (2, A, B))
    for i, src, dst in layers:
        gb = gbs[i]
        for n0 in range(0, N, ch):
            sl = slice(n0, n0 + ch)
            src[:, sl] = jnp.dot(gwt[i], src[:, sl] * dinv_b[:, sl],
                                 preferred_element_type=_F32).astype(_BF)
        for c0 in range(0, _C, _CC):
            cs = slice(c0, c0 + _CC)
            g = src[cs, :]
            agg = (g
                   + mU * roll(g, W) + mD * roll(g, -W)
                   + mL * roll(g, 1) + mR * roll(g, -1))
            dst[cs, :] = dinv_b * agg + gb[cs, :].astype(_BF)

    for n0 in range(0, N, ch):
        sl = slice(n0, n0 + ch)
        t = jnp.maximum(
            jnp.dot(hw1t, B[:, sl], preferred_element_type=_F32) + hb1[:],
            0.0)
        out_ref[:, sl] = jnp.dot(hw2[:].astype(_BF), t.astype(_BF),
                                 preferred_element_type=_F32) + hb2[:]


def _body(f0, f1, f2,
          e0w1, e0b1, e0w2, e0b2,
          e1w1, e1b1, e1w2, e1b2,
          e2w1, e2b1, e2w2, e2b2,
          gw0, gb0, gw1, gb1, gw2, gb2,
          hw1, hb1, hw2, hb2,
          o0, o1, o2, A, B, WT):
    ws = (e0w1, e0w2, e1w1, e1w2, e2w1, e2w2, gw0, gw1, gw2, hw1)
    for i, w in enumerate(ws):
        WT[i, :, :] = w[:].T.astype(_BF)
    gwt = (WT[6], WT[7], WT[8])
    gbs = (gb0, gb1, gb2)
    hw1t = WT[9]
    enc = ((WT[0], e0b1, WT[1], e0b2), (WT[2], e1b1, WT[3], e1b2),
           (WT[4], e2b1, WT[5], e2b2))
    for (H, W), f, o, (w1t, b1, w2t, b2) in zip(_LEVELS, (f0, f1, f2),
                                                (o0, o1, o2), enc):
        N = H * W
        a = A.at[:, :N] if N < _N0 else A
        b = B.at[:, :N] if N < _N0 else B
        _level(H, W, f, w1t, b1, w2t, b2, gwt, gbs, hw1t, hb1, hw2, hb2,
               o, a, b)


@jax.jit
def kernel(feat0, feat1, feat2, e0W1, e0b1, e0W2, e0b2, e1W1, e1b1, e1W2,
           e1b2, e2W1, e2b1, e2W2, e2b2, g0W, g0b, g1W, g1b, g2W, g2b,
           hW1, hb1, hW2, hb2):
    c1 = lambda v: v.reshape(_C, 1)
    args = (
        feat0.reshape(_C, 128 * 128).astype(_BF),
        feat1.reshape(_C, 64 * 64).astype(_BF),
        feat2.reshape(_C, 32 * 32).astype(_BF),
        e0W1, c1(e0b1), e0W2, c1(e0b2),
        e1W1, c1(e1b1), e1W2, c1(e1b2),
        e2W1, c1(e2b1), e2W2, c1(e2b2),
        g0W, c1(g0b), g1W, c1(g1b), g2W, c1(g2b),
        hW1, c1(hb1), hW2.reshape(1, _C), hb2.reshape(1, 1),
    )
    out_shapes = tuple(jax.ShapeDtypeStruct((1, H * W), _F32)
                       for H, W in _LEVELS)
    outs = pl.pallas_call(
        _body,
        out_shape=out_shapes,
        scratch_shapes=[pltpu.VMEM((_C, _N0), _BF),
                        pltpu.VMEM((_C, _N0), _BF),
                        pltpu.VMEM((10, _C, _C), _BF)],
    )(*args)
    return tuple(o.reshape(1, H, W, 1)
                 for o, (H, W) in zip(outs, _LEVELS))
